# baseline (speedup 1.0000x reference)
import functools
import math

import jax
import jax.numpy as jnp
from jax import lax
from jax.experimental import pallas as pl
from jax.experimental.pallas import tpu as pltpu
from jax.experimental.pallas import tpu_sc as plsc

NUM_E = 32
HIDDEN = 768
SEQ = 32
BATCH = 4096

NC, NS = 2, 16
NW = NC * NS
ROWS = BATCH * SEQ
RPW = ROWS // NW
CH = 64
NCH = RPW // CH

_SQRT_HALF = math.sqrt(0.5)


def _mlp_body(tab, w1, b1, g, bb, w2, b2, out):
    x = tab[...]
    h = jnp.dot(x, w1[...], preferred_element_type=jnp.float32,
                precision=lax.Precision.HIGHEST) + b1[...]
    mu = jnp.mean(h, axis=-1, keepdims=True)
    var = jnp.mean((h - mu) ** 2, axis=-1, keepdims=True)
    h = (h - mu) * lax.rsqrt(var + 1e-5) * g[...] + bb[...]
    h = 0.5 * h * (1.0 + lax.erf(h * _SQRT_HALF))
    out[...] = jnp.dot(h, w2[...], preferred_element_type=jnp.float32,
                       precision=lax.Precision.HIGHEST) + b2[...]


def _mlp(tab, w1, b1, g, bb, w2, b2):
    return pl.pallas_call(
        _mlp_body,
        out_shape=jax.ShapeDtypeStruct((NUM_E, HIDDEN), jnp.float32),
    )(tab, w1, b1, g, bb, w2, b2)


@functools.cache
def _make_expand():
    mesh = plsc.VectorSubcoreMesh(core_axis_name="c", subcore_axis_name="s",
                                  num_cores=NC, num_subcores=NS)

    @functools.partial(
        pl.kernel,
        out_type=jax.ShapeDtypeStruct((ROWS, HIDDEN), jnp.float32),
        mesh=mesh,
        scratch_types=[
            pltpu.VMEM((RPW,), jnp.int32),
            pltpu.VMEM((CH, HIDDEN), jnp.float32),
            pltpu.SemaphoreType.DMA,
        ],
    )
    def _expand(ptab_hbm, idx_hbm, out_hbm, idx_v, rows_v, sem):
        wid = lax.axis_index("s") * NC + lax.axis_index("c")
        base = wid * RPW
        pltpu.sync_copy(idx_hbm.at[pl.ds(base, RPW)], idx_v)

        def chunk(c, carry):
            pltpu.async_copy(
                ptab_hbm.at[idx_v.at[pl.ds(c * CH, CH)]], rows_v, sem).wait()
            pltpu.sync_copy(rows_v, out_hbm.at[pl.ds(base + c * CH, CH)])
            return carry

        lax.fori_loop(0, NCH, chunk, 0)

    return _expand


BB = 128
NB = BATCH // BB


def _fused_body(ids_ref, tab_ref, w1_ref, b1_ref, g_ref, bb_ref, w2_ref,
                b2_ref, out_ref, ptab_scr):
    i = pl.program_id(0)

    @pl.when(i == 0)
    def _():
        _mlp_body(tab_ref, w1_ref, b1_ref, g_ref, bb_ref, w2_ref, b2_ref,
                  ptab_scr)

    ids_blk = ids_ref[0, pl.ds(i * BB, BB)]
    onehot = (ids_blk[:, None] == lax.broadcasted_iota(
        jnp.int32, (BB, NUM_E), 1)).astype(jnp.float32)
    g = jnp.dot(onehot, ptab_scr[...], preferred_element_type=jnp.float32,
                precision=lax.Precision.HIGHEST)
    for s in range(SEQ):
        out_ref[:, s, :] = g


def _fused(ids, tab, w1, b1, g, bb, w2, b2):
    whole = lambda shape: pl.BlockSpec(shape, lambda i: tuple(0 for _ in shape))
    return pl.pallas_call(
        _fused_body,
        grid=(NB,),
        in_specs=[
            whole((1, BATCH)),
            whole((NUM_E, HIDDEN)),
            whole((HIDDEN, HIDDEN)),
            whole((1, HIDDEN)),
            whole((1, HIDDEN)),
            whole((1, HIDDEN)),
            whole((HIDDEN, HIDDEN)),
            whole((1, HIDDEN)),
        ],
        out_specs=pl.BlockSpec((BB, SEQ, HIDDEN), lambda i: (i, 0, 0)),
        out_shape=jax.ShapeDtypeStruct((BATCH, SEQ, HIDDEN), jnp.float32),
        scratch_shapes=[pltpu.VMEM((NUM_E, HIDDEN), jnp.float32)],
    )(ids.reshape(1, BATCH), tab, w1, b1, g, bb, w2, b2)


@functools.cache
def _make_expand_pipe(rows_total):
    rpw = rows_total // NW
    nch = rpw // CH
    n2 = nch // 2
    mesh = plsc.VectorSubcoreMesh(core_axis_name="c", subcore_axis_name="s",
                                  num_cores=NC, num_subcores=NS)

    @functools.partial(
        pl.kernel,
        out_type=jax.ShapeDtypeStruct((rows_total, HIDDEN), jnp.float32),
        mesh=mesh,
        scratch_types=[
            pltpu.VMEM((rpw,), jnp.int32),
            pltpu.VMEM((CH, HIDDEN), jnp.float32),
            pltpu.VMEM((CH, HIDDEN), jnp.float32),
            pltpu.SemaphoreType.DMA,
            pltpu.SemaphoreType.DMA,
            pltpu.SemaphoreType.DMA,
            pltpu.SemaphoreType.DMA,
        ],
    )
    def _expand(ptab_hbm, idx_hbm, out_hbm, idx_v, rows0, rows1,
                gs0, gs1, ws0, ws1):
        wid = lax.axis_index("s") * NC + lax.axis_index("c")
        base = wid * rpw
        pltpu.sync_copy(idx_hbm.at[pl.ds(base, rpw)], idx_v)
        bufs = (rows0, rows1)
        gsems = (gs0, gs1)
        wsems = (ws0, ws1)

        def start_gather(c, k):
            pltpu.async_copy(ptab_hbm.at[idx_v.at[pl.ds(c * CH, CH)]],
                             bufs[k], gsems[k])

        def wait_gather(c, k):
            pltpu.make_async_copy(ptab_hbm.at[idx_v.at[pl.ds(c * CH, CH)]],
                                  bufs[k], gsems[k]).wait()

        def start_write(c, k):
            pltpu.async_copy(bufs[k], out_hbm.at[pl.ds(base + c * CH, CH)],
                             wsems[k])

        def wait_write(c, k):
            pltpu.make_async_copy(bufs[k], out_hbm.at[pl.ds(base + c * CH, CH)],
                                  wsems[k]).wait()

        start_gather(0, 0)
        start_gather(1, 1)

        def body(i, carry):
            a = 2 * i
            b = a + 1
            wait_gather(a, 0)
            start_write(a, 0)
            wait_gather(b, 1)
            start_write(b, 1)

            @pl.when(i + 1 < n2)
            def _():
                wait_write(a, 0)
                start_gather(a + 2, 0)
                wait_write(b, 1)
                start_gather(b + 2, 1)

            return carry

        lax.fori_loop(0, n2, body, 0)
        wait_write(nch - 2, 0)
        wait_write(nch - 1, 1)

    return _expand


SB = 1024
TB = BATCH - SB


def _expand_tc(ptab, ids, tb):
    return pl.pallas_call(
        functools.partial(_expand_tc_body, tb=tb),
        grid=(tb // BB,),
        in_specs=[
            pl.BlockSpec((1, tb), lambda i: (0, 0)),
            pl.BlockSpec((NUM_E, HIDDEN), lambda i: (0, 0)),
        ],
        out_specs=pl.BlockSpec((BB, SEQ, HIDDEN), lambda i: (i, 0, 0)),
        out_shape=jax.ShapeDtypeStruct((tb, SEQ, HIDDEN), jnp.float32),
    )(ids.reshape(1, tb), ptab)


def _expand_tc_body(ids_ref, ptab_ref, out_ref, *, tb):
    i = pl.program_id(0)
    ids_blk = ids_ref[0, pl.ds(i * BB, BB)]
    onehot = (ids_blk[:, None] == lax.broadcasted_iota(
        jnp.int32, (BB, NUM_E), 1)).astype(jnp.float32)
    g = jnp.dot(onehot, ptab_ref[...], preferred_element_type=jnp.float32,
                precision=lax.Precision.HIGHEST)
    for s in range(SEQ):
        out_ref[:, s, :] = g


def kernel(emotion_ids, embed_table, W1, b1, ln_g, ln_b, W2, b2):
    ids = emotion_ids.astype(jnp.int32)
    ptab = _mlp(embed_table, W1, b1.reshape(1, HIDDEN), ln_g.reshape(1, HIDDEN),
                ln_b.reshape(1, HIDDEN), W2, b2.reshape(1, HIDDEN))
    tc_out = _expand_tc(ptab, ids[:TB], TB)
    idx_sc = jnp.repeat(ids[TB:], SEQ)
    sc_out = _make_expand_pipe(SB * SEQ)(ptab, idx_sc)
    return jnp.concatenate([tc_out, sc_out.reshape(SB, SEQ, HIDDEN)], axis=0)


# speedup vs baseline: 4.4312x; 4.4312x over previous
import functools
import math

import jax
import jax.numpy as jnp
from jax import lax
from jax.experimental import pallas as pl
from jax.experimental.pallas import tpu as pltpu
from jax.experimental.pallas import tpu_sc as plsc

NUM_E = 32
HIDDEN = 768
SEQ = 32
BATCH = 4096

NC, NS = 2, 16
NW = NC * NS
ROWS = BATCH * SEQ
RPW = ROWS // NW
CH = 64
NCH = RPW // CH

_SQRT_HALF = math.sqrt(0.5)


def _mlp_body(tab, w1, b1, g, bb, w2, b2, out):
    x = tab[...]
    h = jnp.dot(x, w1[...], preferred_element_type=jnp.float32,
                precision=lax.Precision.HIGHEST) + b1[...]
    mu = jnp.mean(h, axis=-1, keepdims=True)
    var = jnp.mean((h - mu) ** 2, axis=-1, keepdims=True)
    h = (h - mu) * lax.rsqrt(var + 1e-5) * g[...] + bb[...]
    h = 0.5 * h * (1.0 + lax.erf(h * _SQRT_HALF))
    out[...] = jnp.dot(h, w2[...], preferred_element_type=jnp.float32,
                       precision=lax.Precision.HIGHEST) + b2[...]


def _mlp(tab, w1, b1, g, bb, w2, b2):
    return pl.pallas_call(
        _mlp_body,
        out_shape=jax.ShapeDtypeStruct((NUM_E, HIDDEN), jnp.float32),
    )(tab, w1, b1, g, bb, w2, b2)


@functools.cache
def _make_expand():
    mesh = plsc.VectorSubcoreMesh(core_axis_name="c", subcore_axis_name="s",
                                  num_cores=NC, num_subcores=NS)

    @functools.partial(
        pl.kernel,
        out_type=jax.ShapeDtypeStruct((ROWS, HIDDEN), jnp.float32),
        mesh=mesh,
        scratch_types=[
            pltpu.VMEM((RPW,), jnp.int32),
            pltpu.VMEM((CH, HIDDEN), jnp.float32),
            pltpu.SemaphoreType.DMA,
        ],
    )
    def _expand(ptab_hbm, idx_hbm, out_hbm, idx_v, rows_v, sem):
        wid = lax.axis_index("s") * NC + lax.axis_index("c")
        base = wid * RPW
        pltpu.sync_copy(idx_hbm.at[pl.ds(base, RPW)], idx_v)

        def chunk(c, carry):
            pltpu.async_copy(
                ptab_hbm.at[idx_v.at[pl.ds(c * CH, CH)]], rows_v, sem).wait()
            pltpu.sync_copy(rows_v, out_hbm.at[pl.ds(base + c * CH, CH)])
            return carry

        lax.fori_loop(0, NCH, chunk, 0)

    return _expand


BB = 256
NB = BATCH // BB


def _fused_body(ids_ref, tab_ref, w1_ref, b1_ref, g_ref, bb_ref, w2_ref,
                b2_ref, out_ref, ptab_scr):
    i = pl.program_id(0)

    @pl.when(i == 0)
    def _():
        _mlp_body(tab_ref, w1_ref, b1_ref, g_ref, bb_ref, w2_ref, b2_ref,
                  ptab_scr)

    ids_blk = ids_ref[0, pl.ds(i * BB, BB)]
    onehot = (ids_blk[:, None] == lax.broadcasted_iota(
        jnp.int32, (BB, NUM_E), 1)).astype(jnp.float32)
    g = jnp.dot(onehot, ptab_scr[...], preferred_element_type=jnp.float32,
                precision=lax.Precision.HIGHEST)
    for s in range(SEQ):
        out_ref[:, s, :] = g


def _fused(ids, tab, w1, b1, g, bb, w2, b2):
    whole = lambda shape: pl.BlockSpec(shape, lambda i: tuple(0 for _ in shape))
    return pl.pallas_call(
        _fused_body,
        grid=(NB,),
        in_specs=[
            whole((1, BATCH)),
            whole((NUM_E, HIDDEN)),
            whole((HIDDEN, HIDDEN)),
            whole((1, HIDDEN)),
            whole((1, HIDDEN)),
            whole((1, HIDDEN)),
            whole((HIDDEN, HIDDEN)),
            whole((1, HIDDEN)),
        ],
        out_specs=pl.BlockSpec((BB, SEQ, HIDDEN), lambda i: (i, 0, 0)),
        out_shape=jax.ShapeDtypeStruct((BATCH, SEQ, HIDDEN), jnp.float32),
        scratch_shapes=[pltpu.VMEM((NUM_E, HIDDEN), jnp.float32)],
    )(ids.reshape(1, BATCH), tab, w1, b1, g, bb, w2, b2)


@functools.cache
def _make_expand_pipe(rows_total):
    rpw = rows_total // NW
    nch = rpw // CH
    n2 = nch // 2
    mesh = plsc.VectorSubcoreMesh(core_axis_name="c", subcore_axis_name="s",
                                  num_cores=NC, num_subcores=NS)

    @functools.partial(
        pl.kernel,
        out_type=jax.ShapeDtypeStruct((rows_total, HIDDEN), jnp.float32),
        mesh=mesh,
        scratch_types=[
            pltpu.VMEM((rpw,), jnp.int32),
            pltpu.VMEM((CH, HIDDEN), jnp.float32),
            pltpu.VMEM((CH, HIDDEN), jnp.float32),
            pltpu.SemaphoreType.DMA,
            pltpu.SemaphoreType.DMA,
            pltpu.SemaphoreType.DMA,
            pltpu.SemaphoreType.DMA,
        ],
    )
    def _expand(ptab_hbm, idx_hbm, out_hbm, idx_v, rows0, rows1,
                gs0, gs1, ws0, ws1):
        wid = lax.axis_index("s") * NC + lax.axis_index("c")
        base = wid * rpw
        pltpu.sync_copy(idx_hbm.at[pl.ds(base, rpw)], idx_v)
        bufs = (rows0, rows1)
        gsems = (gs0, gs1)
        wsems = (ws0, ws1)

        def start_gather(c, k):
            pltpu.async_copy(ptab_hbm.at[idx_v.at[pl.ds(c * CH, CH)]],
                             bufs[k], gsems[k])

        def wait_gather(c, k):
            pltpu.make_async_copy(ptab_hbm.at[idx_v.at[pl.ds(c * CH, CH)]],
                                  bufs[k], gsems[k]).wait()

        def start_write(c, k):
            pltpu.async_copy(bufs[k], out_hbm.at[pl.ds(base + c * CH, CH)],
                             wsems[k])

        def wait_write(c, k):
            pltpu.make_async_copy(bufs[k], out_hbm.at[pl.ds(base + c * CH, CH)],
                                  wsems[k]).wait()

        start_gather(0, 0)
        start_gather(1, 1)

        def body(i, carry):
            a = 2 * i
            b = a + 1
            wait_gather(a, 0)
            start_write(a, 0)
            wait_gather(b, 1)
            start_write(b, 1)

            @pl.when(i + 1 < n2)
            def _():
                wait_write(a, 0)
                start_gather(a + 2, 0)
                wait_write(b, 1)
                start_gather(b + 2, 1)

            return carry

        lax.fori_loop(0, n2, body, 0)
        wait_write(nch - 2, 0)
        wait_write(nch - 1, 1)

    return _expand


SB = 1024
TB = BATCH - SB


def _expand_tc(ptab, ids, tb):
    return pl.pallas_call(
        functools.partial(_expand_tc_body, tb=tb),
        grid=(tb // BB,),
        in_specs=[
            pl.BlockSpec((1, tb), lambda i: (0, 0)),
            pl.BlockSpec((NUM_E, HIDDEN), lambda i: (0, 0)),
        ],
        out_specs=pl.BlockSpec((BB, SEQ, HIDDEN), lambda i: (i, 0, 0)),
        out_shape=jax.ShapeDtypeStruct((tb, SEQ, HIDDEN), jnp.float32),
    )(ids.reshape(1, tb), ptab)


def _expand_tc_body(ids_ref, ptab_ref, out_ref, *, tb):
    i = pl.program_id(0)
    ids_blk = ids_ref[0, pl.ds(i * BB, BB)]
    onehot = (ids_blk[:, None] == lax.broadcasted_iota(
        jnp.int32, (BB, NUM_E), 1)).astype(jnp.float32)
    g = jnp.dot(onehot, ptab_ref[...], preferred_element_type=jnp.float32,
                precision=lax.Precision.HIGHEST)
    for s in range(SEQ):
        out_ref[:, s, :] = g


def kernel(emotion_ids, embed_table, W1, b1, ln_g, ln_b, W2, b2):
    ids = emotion_ids.astype(jnp.int32)
    return _fused(ids, embed_table, W1, b1.reshape(1, HIDDEN),
                  ln_g.reshape(1, HIDDEN), ln_b.reshape(1, HIDDEN),
                  W2, b2.reshape(1, HIDDEN))


# speedup vs baseline: 4.6530x; 1.0501x over previous
import functools
import math

import jax
import jax.numpy as jnp
from jax import lax
from jax.experimental import pallas as pl
from jax.experimental.pallas import tpu as pltpu
from jax.experimental.pallas import tpu_sc as plsc

NUM_E = 32
HIDDEN = 768
SEQ = 32
BATCH = 4096

NC, NS = 2, 16
NW = NC * NS
ROWS = BATCH * SEQ
RPW = ROWS // NW
CH = 64
NCH = RPW // CH

_SQRT_HALF = math.sqrt(0.5)


def _mlp_body(tab, w1, b1, g, bb, w2, b2, out):
    x = tab[...]
    h = jnp.dot(x, w1[...], preferred_element_type=jnp.float32,
                precision=lax.Precision.HIGHEST) + b1[...]
    mu = jnp.mean(h, axis=-1, keepdims=True)
    var = jnp.mean((h - mu) ** 2, axis=-1, keepdims=True)
    h = (h - mu) * lax.rsqrt(var + 1e-5) * g[...] + bb[...]
    h = 0.5 * h * (1.0 + lax.erf(h * _SQRT_HALF))
    out[...] = jnp.dot(h, w2[...], preferred_element_type=jnp.float32,
                       precision=lax.Precision.HIGHEST) + b2[...]


def _mlp(tab, w1, b1, g, bb, w2, b2):
    return pl.pallas_call(
        _mlp_body,
        out_shape=jax.ShapeDtypeStruct((NUM_E, HIDDEN), jnp.float32),
    )(tab, w1, b1, g, bb, w2, b2)


@functools.cache
def _make_expand():
    mesh = plsc.VectorSubcoreMesh(core_axis_name="c", subcore_axis_name="s",
                                  num_cores=NC, num_subcores=NS)

    @functools.partial(
        pl.kernel,
        out_type=jax.ShapeDtypeStruct((ROWS, HIDDEN), jnp.float32),
        mesh=mesh,
        scratch_types=[
            pltpu.VMEM((RPW,), jnp.int32),
            pltpu.VMEM((CH, HIDDEN), jnp.float32),
            pltpu.SemaphoreType.DMA,
        ],
    )
    def _expand(ptab_hbm, idx_hbm, out_hbm, idx_v, rows_v, sem):
        wid = lax.axis_index("s") * NC + lax.axis_index("c")
        base = wid * RPW
        pltpu.sync_copy(idx_hbm.at[pl.ds(base, RPW)], idx_v)

        def chunk(c, carry):
            pltpu.async_copy(
                ptab_hbm.at[idx_v.at[pl.ds(c * CH, CH)]], rows_v, sem).wait()
            pltpu.sync_copy(rows_v, out_hbm.at[pl.ds(base + c * CH, CH)])
            return carry

        lax.fori_loop(0, NCH, chunk, 0)

    return _expand


BB = 128
NB = BATCH // BB


def _fused_body(ids_ref, tab_ref, w1_ref, b1_ref, g_ref, bb_ref, w2_ref,
                b2_ref, out_ref, ptab_scr, g0, g1, sem0, sem1):
    i = pl.program_id(0)

    @pl.when(i == 0)
    def _():
        _mlp_body(tab_ref, w1_ref, b1_ref, g_ref, bb_ref, w2_ref, b2_ref,
                  ptab_scr)

    ids_blk = ids_ref[0, pl.ds(i * BB, BB)]
    onehot = (ids_blk[:, None] == lax.broadcasted_iota(
        jnp.int32, (BB, NUM_E), 1)).astype(jnp.float32)
    g = jnp.dot(onehot, ptab_scr[...], preferred_element_type=jnp.float32,
                precision=lax.Precision.HIGHEST)[:, None, :]

    def run(buf, sem):
        @pl.when(i >= 2)
        def _():
            for s in range(SEQ):
                pltpu.make_async_copy(
                    buf, out_ref.at[pl.ds((i - 2) * BB, BB),
                                    pl.ds(s, 1), :], sem).wait()

        buf[...] = g
        for s in range(SEQ):
            pltpu.async_copy(
                buf, out_ref.at[pl.ds(i * BB, BB), pl.ds(s, 1), :], sem)

    @pl.when(i % 2 == 0)
    def _():
        run(g0, sem0)

    @pl.when(i % 2 == 1)
    def _():
        run(g1, sem1)

    @pl.when(i == NB - 1)
    def _():
        for s in range(SEQ):
            pltpu.make_async_copy(
                g0, out_ref.at[pl.ds(0, BB), pl.ds(s, 1), :], sem0).wait()
            pltpu.make_async_copy(
                g1, out_ref.at[pl.ds(0, BB), pl.ds(s, 1), :], sem1).wait()


def _fused(ids, tab, w1, b1, g, bb, w2, b2):
    whole = lambda shape: pl.BlockSpec(shape, lambda i: tuple(0 for _ in shape))
    return pl.pallas_call(
        _fused_body,
        grid=(NB,),
        in_specs=[
            whole((1, BATCH)),
            whole((NUM_E, HIDDEN)),
            whole((HIDDEN, HIDDEN)),
            whole((1, HIDDEN)),
            whole((1, HIDDEN)),
            whole((1, HIDDEN)),
            whole((HIDDEN, HIDDEN)),
            whole((1, HIDDEN)),
        ],
        out_specs=pl.BlockSpec(memory_space=pl.ANY),
        out_shape=jax.ShapeDtypeStruct((BATCH, SEQ, HIDDEN), jnp.float32),
        scratch_shapes=[
            pltpu.VMEM((NUM_E, HIDDEN), jnp.float32),
            pltpu.VMEM((BB, 1, HIDDEN), jnp.float32),
            pltpu.VMEM((BB, 1, HIDDEN), jnp.float32),
            pltpu.SemaphoreType.DMA,
            pltpu.SemaphoreType.DMA,
        ],
    )(ids.reshape(1, BATCH), tab, w1, b1, g, bb, w2, b2)


@functools.cache
def _make_expand_pipe(rows_total):
    rpw = rows_total // NW
    nch = rpw // CH
    n2 = nch // 2
    mesh = plsc.VectorSubcoreMesh(core_axis_name="c", subcore_axis_name="s",
                                  num_cores=NC, num_subcores=NS)

    @functools.partial(
        pl.kernel,
        out_type=jax.ShapeDtypeStruct((rows_total, HIDDEN), jnp.float32),
        mesh=mesh,
        scratch_types=[
            pltpu.VMEM((rpw,), jnp.int32),
            pltpu.VMEM((CH, HIDDEN), jnp.float32),
            pltpu.VMEM((CH, HIDDEN), jnp.float32),
            pltpu.SemaphoreType.DMA,
            pltpu.SemaphoreType.DMA,
            pltpu.SemaphoreType.DMA,
            pltpu.SemaphoreType.DMA,
        ],
    )
    def _expand(ptab_hbm, idx_hbm, out_hbm, idx_v, rows0, rows1,
                gs0, gs1, ws0, ws1):
        wid = lax.axis_index("s") * NC + lax.axis_index("c")
        base = wid * rpw
        pltpu.sync_copy(idx_hbm.at[pl.ds(base, rpw)], idx_v)
        bufs = (rows0, rows1)
        gsems = (gs0, gs1)
        wsems = (ws0, ws1)

        def start_gather(c, k):
            pltpu.async_copy(ptab_hbm.at[idx_v.at[pl.ds(c * CH, CH)]],
                             bufs[k], gsems[k])

        def wait_gather(c, k):
            pltpu.make_async_copy(ptab_hbm.at[idx_v.at[pl.ds(c * CH, CH)]],
                                  bufs[k], gsems[k]).wait()

        def start_write(c, k):
            pltpu.async_copy(bufs[k], out_hbm.at[pl.ds(base + c * CH, CH)],
                             wsems[k])

        def wait_write(c, k):
            pltpu.make_async_copy(bufs[k], out_hbm.at[pl.ds(base + c * CH, CH)],
                                  wsems[k]).wait()

        start_gather(0, 0)
        start_gather(1, 1)

        def body(i, carry):
            a = 2 * i
            b = a + 1
            wait_gather(a, 0)
            start_write(a, 0)
            wait_gather(b, 1)
            start_write(b, 1)

            @pl.when(i + 1 < n2)
            def _():
                wait_write(a, 0)
                start_gather(a + 2, 0)
                wait_write(b, 1)
                start_gather(b + 2, 1)

            return carry

        lax.fori_loop(0, n2, body, 0)
        wait_write(nch - 2, 0)
        wait_write(nch - 1, 1)

    return _expand


SB = 1024
TB = BATCH - SB


def _expand_tc(ptab, ids, tb):
    return pl.pallas_call(
        functools.partial(_expand_tc_body, tb=tb),
        grid=(tb // BB,),
        in_specs=[
            pl.BlockSpec((1, tb), lambda i: (0, 0)),
            pl.BlockSpec((NUM_E, HIDDEN), lambda i: (0, 0)),
        ],
        out_specs=pl.BlockSpec((BB, SEQ, HIDDEN), lambda i: (i, 0, 0)),
        out_shape=jax.ShapeDtypeStruct((tb, SEQ, HIDDEN), jnp.float32),
    )(ids.reshape(1, tb), ptab)


def _expand_tc_body(ids_ref, ptab_ref, out_ref, *, tb):
    i = pl.program_id(0)
    ids_blk = ids_ref[0, pl.ds(i * BB, BB)]
    onehot = (ids_blk[:, None] == lax.broadcasted_iota(
        jnp.int32, (BB, NUM_E), 1)).astype(jnp.float32)
    g = jnp.dot(onehot, ptab_ref[...], preferred_element_type=jnp.float32,
                precision=lax.Precision.HIGHEST)
    for s in range(SEQ):
        out_ref[:, s, :] = g


def kernel(emotion_ids, embed_table, W1, b1, ln_g, ln_b, W2, b2):
    ids = emotion_ids.astype(jnp.int32)
    return _fused(ids, embed_table, W1, b1.reshape(1, HIDDEN),
                  ln_g.reshape(1, HIDDEN), ln_b.reshape(1, HIDDEN),
                  W2, b2.reshape(1, HIDDEN))
